# baseline (speedup 1.0000x reference)
import functools

import jax
import jax.numpy as jnp
from jax import lax
from jax.experimental import pallas as pl
from jax.experimental.pallas import tpu as pltpu
from jax.experimental.pallas import tpu_sc as plsc

_INFO = plsc.get_sparse_core_info()
_NC = _INFO.num_cores
_NS = _INFO.num_subcores
_NW = _NC * _NS

_CHUNK = 128


def _gather_fn(n_chunks, n_rows, hidden):
    mesh = plsc.VectorSubcoreMesh(core_axis_name="c", subcore_axis_name="s")
    rows_per_w = n_chunks * _CHUNK

    @functools.partial(
        pl.kernel,
        mesh=mesh,
        out_type=jax.ShapeDtypeStruct((n_rows, hidden), jnp.float32),
        scratch_types=[
            pltpu.VMEM((n_chunks, _CHUNK), jnp.int32),
            pltpu.VMEM((_CHUNK, hidden), jnp.float32),
            pltpu.SemaphoreType.DMA,
        ],
        compiler_params=pltpu.CompilerParams(use_tc_tiling_on_sc=False),
    )
    def k(idx_hbm, table_hbm, out_hbm, idx_v, rows_v, sem):
        wid = lax.axis_index("s") * _NC + lax.axis_index("c")
        base = wid * rows_per_w
        pltpu.sync_copy(idx_hbm.at[wid], idx_v)

        def chunk(j, carry):
            pltpu.async_copy(table_hbm.at[idx_v.at[j]], rows_v, sem).wait()
            pltpu.sync_copy(rows_v, out_hbm.at[pl.ds(base + j * _CHUNK, _CHUNK)])
            return carry

        lax.fori_loop(0, n_chunks, chunk, 0)

    return k


def kernel(input_ids, word_embeddings):
    batch, seq = input_ids.shape
    vocab, hidden = word_embeddings.shape
    n_rows = batch * seq
    assert n_rows % (_NW * _CHUNK) == 0
    n_chunks = n_rows // (_NW * _CHUNK)
    ids = input_ids.reshape(_NW, n_chunks, _CHUNK).astype(jnp.int32)
    out = _gather_fn(n_chunks, n_rows, hidden)(ids, word_embeddings)
    return out.reshape(batch, seq, hidden)


# speedup vs baseline: 1.1139x; 1.1139x over previous
import functools

import jax
import jax.numpy as jnp
from jax import lax
from jax.experimental import pallas as pl
from jax.experimental.pallas import tpu as pltpu
from jax.experimental.pallas import tpu_sc as plsc

_INFO = plsc.get_sparse_core_info()
_NC = _INFO.num_cores
_NS = _INFO.num_subcores
_NW = _NC * _NS

_CHUNK = 128
_NBUF = 4


def _gather_fn(n_chunks, n_rows, hidden):
    mesh = plsc.VectorSubcoreMesh(core_axis_name="c", subcore_axis_name="s")
    rows_per_w = n_chunks * _CHUNK
    n_main = n_chunks - _NBUF
    assert n_main >= 0 and n_main % _NBUF == 0

    @functools.partial(
        pl.kernel,
        mesh=mesh,
        out_type=jax.ShapeDtypeStruct((n_rows, hidden), jnp.float32),
        scratch_types=[
            pltpu.VMEM((n_chunks, _CHUNK), jnp.int32),
            pltpu.VMEM((_NBUF, _CHUNK, hidden), jnp.float32),
            pltpu.SemaphoreType.DMA((_NBUF,)),
            pltpu.SemaphoreType.DMA((_NBUF,)),
        ],
        compiler_params=pltpu.CompilerParams(use_tc_tiling_on_sc=False),
    )
    def k(idx_hbm, table_hbm, out_hbm, idx_v, rows_v, gsem, ssem):
        wid = lax.axis_index("s") * _NC + lax.axis_index("c")
        base = wid * rows_per_w
        pltpu.sync_copy(idx_hbm.at[wid], idx_v)

        def gather_start(j, b):
            pltpu.async_copy(table_hbm.at[idx_v.at[j]], rows_v.at[b], gsem.at[b])

        def gather_wait(j, b):
            pltpu.make_async_copy(
                table_hbm.at[idx_v.at[j]], rows_v.at[b], gsem.at[b]).wait()

        def store_start(j, b):
            pltpu.async_copy(
                rows_v.at[b], out_hbm.at[pl.ds(base + j * _CHUNK, _CHUNK)],
                ssem.at[b])

        def store_wait(j, b):
            pltpu.make_async_copy(
                rows_v.at[b], out_hbm.at[pl.ds(base + j * _CHUNK, _CHUNK)],
                ssem.at[b]).wait()

        for b in range(_NBUF):
            gather_start(b, b)

        def outer(g, carry):
            j0 = g * _NBUF
            for b in range(_NBUF):
                j = j0 + b
                gather_wait(j, b)
                store_start(j, b)
            for b in range(_NBUF):
                j = j0 + b
                store_wait(j, b)
                gather_start(j + _NBUF, b)
            return carry

        lax.fori_loop(0, n_main // _NBUF, outer, 0)

        for b in range(_NBUF):
            j = n_main + b
            gather_wait(j, b)
            store_start(j, b)
        for b in range(_NBUF):
            store_wait(n_main + b, b)

    return k


def kernel(input_ids, word_embeddings):
    batch, seq = input_ids.shape
    vocab, hidden = word_embeddings.shape
    n_rows = batch * seq
    assert n_rows % (_NW * _CHUNK) == 0
    n_chunks = n_rows // (_NW * _CHUNK)
    ids = input_ids.reshape(_NW, n_chunks, _CHUNK).astype(jnp.int32)
    out = _gather_fn(n_chunks, n_rows, hidden)(ids, word_embeddings)
    return out.reshape(batch, seq, hidden)
